# Initial kernel scaffold; baseline (speedup 1.0000x reference)
import functools

import jax
import jax.numpy as jnp
from jax import lax
from jax.experimental import pallas as pl
from jax.experimental.pallas import tpu as pltpu
from jax.experimental.pallas import tpu_sc as plsc

_N = 10000
_D = 128
_E = 320000
_NC = 2
_NS = 16
_NW = _NC * _NS
_BE = 128
_NCH = 80
_EPW = _NCH * _BE
_EP = _NW * _EPW
_ZR = 626
_NPAD = _NS * _ZR
_OR = 625


def _segment_sum_partials(h, srcr, dstr, zeros):
    mesh = plsc.VectorSubcoreMesh(core_axis_name="c", subcore_axis_name="s")

    @functools.partial(
        pl.kernel,
        out_type=jax.ShapeDtypeStruct((_NC, _N, _D), jnp.float32),
        mesh=mesh,
        scratch_types=[
            pltpu.VMEM((_NCH, _BE), jnp.int32),
            pltpu.VMEM((_NCH, _BE), jnp.int32),
            pltpu.VMEM((_BE, _D), jnp.float32),
            pltpu.VMEM_SHARED((_NPAD, _D), jnp.float32),
            pltpu.SemaphoreType.DMA,
        ],
    )
    def k(h_hbm, src_hbm, dst_hbm, zero_hbm, out_hbm, src_v, dst_v, rows_v, agg_s, sem):
        c = lax.axis_index("c")
        s = lax.axis_index("s")
        wid = s * _NC + c
        pltpu.sync_copy(zero_hbm.at[pl.ds(s * _ZR, _ZR)], agg_s.at[pl.ds(s * _ZR, _ZR)])
        pltpu.sync_copy(src_hbm.at[wid], src_v)
        pltpu.sync_copy(dst_hbm.at[wid], dst_v)
        plsc.subcore_barrier()

        @pl.loop(0, _NCH)
        def _(j):
            pltpu.async_copy(h_hbm.at[src_v.at[j]], rows_v, sem).wait()
            pltpu.sync_copy(rows_v, agg_s.at[dst_v.at[j]], add=True)

        plsc.subcore_barrier()
        pltpu.sync_copy(agg_s.at[pl.ds(s * _OR, _OR)],
                        out_hbm.at[c, pl.ds(s * _OR, _OR)])

    return k(h, srcr, dstr, zeros)


def _mlp(h, p0, p1, Wa, ba, Wb, bb, relu_out):
    R = 1000

    def body(h_ref, p0_ref, p1_ref, wa_ref, ba_ref, wb_ref, bb_ref, o_ref):
        m = h_ref[...] + p0_ref[...] + p1_ref[...]
        hid = jnp.dot(m, wa_ref[...], preferred_element_type=jnp.float32) + ba_ref[...]
        hid = jnp.maximum(hid, 0.0)
        o = jnp.dot(hid, wb_ref[...], preferred_element_type=jnp.float32) + bb_ref[...]
        if relu_out:
            o = jnp.maximum(o, 0.0)
        o_ref[...] = o

    bs_rows = pl.BlockSpec((R, _D), lambda i: (i, 0))
    bs_w = pl.BlockSpec((_D, _D), lambda i: (0, 0))
    bs_b = pl.BlockSpec((1, _D), lambda i: (0, 0))
    return pl.pallas_call(
        body,
        grid=(_N // R,),
        in_specs=[bs_rows, bs_rows, bs_rows, bs_w, bs_b, bs_w, bs_b],
        out_specs=bs_rows,
        out_shape=jax.ShapeDtypeStruct((_N, _D), jnp.float32),
    )(h, p0, p1, Wa, ba.reshape(1, _D), Wb, bb.reshape(1, _D))


def kernel(x, edge_index, W0a, b0a, W0b, b0b, W1a, b1a, W1b, b1b, W2a, b2a, W2b, b2b):
    src = edge_index[0]
    dst = edge_index[1]
    pad = _EP - _E
    srcr = jnp.concatenate([src, jnp.zeros((pad,), jnp.int32)]).reshape(_NW, _NCH, _BE)
    dstr = jnp.concatenate([dst, jnp.full((pad,), _N, jnp.int32)]).reshape(_NW, _NCH, _BE)
    zeros = jnp.zeros((_NPAD, _D), jnp.float32)

    h = x
    for Wa, ba, Wb, bb, relu_out in (
        (W0a, b0a, W0b, b0b, True),
        (W1a, b1a, W1b, b1b, True),
        (W2a, b2a, W2b, b2b, False),
    ):
        parts = _segment_sum_partials(h, srcr, dstr, zeros)
        h = _mlp(h, parts[0], parts[1], Wa, ba, Wb, bb, relu_out)
    return h


# speedup vs baseline: 2.7341x; 2.7341x over previous
import functools

import jax
import jax.numpy as jnp
from jax import lax
from jax.experimental import pallas as pl
from jax.experimental.pallas import tpu as pltpu
from jax.experimental.pallas import tpu_sc as plsc

_N = 10000
_D = 128
_E = 320000
_NC = 2
_NS = 16
_NW = _NC * _NS
_BE = 128
_NCH = 80
_EPW = _NCH * _BE
_EP = _NW * _EPW
_ZR = 640
_NPAD = _NS * _ZR


def _segment_sum_partials(h, srcr, dstr, zeros):
    mesh = plsc.VectorSubcoreMesh(core_axis_name="c", subcore_axis_name="s")

    @functools.partial(
        pl.kernel,
        out_type=jax.ShapeDtypeStruct((_NC, _NPAD, _D), jnp.float32),
        mesh=mesh,
        scratch_types=[
            pltpu.VMEM((_NCH, _BE), jnp.int32),
            pltpu.VMEM((_NCH, _BE), jnp.int32),
            pltpu.VMEM((_BE, _D), jnp.float32),
            pltpu.VMEM_SHARED((_NPAD, _D), jnp.float32),
            pltpu.SemaphoreType.DMA,
        ],
    )
    def k(h_hbm, src_hbm, dst_hbm, zero_hbm, out_hbm, src_v, dst_v, rows_v, agg_s, sem):
        c = lax.axis_index("c")
        s = lax.axis_index("s")
        wid = s * _NC + c
        pltpu.sync_copy(zero_hbm.at[pl.ds(s * _ZR, _ZR)], agg_s.at[pl.ds(s * _ZR, _ZR)])
        pltpu.sync_copy(src_hbm.at[wid], src_v)
        pltpu.sync_copy(dst_hbm.at[wid], dst_v)
        plsc.subcore_barrier()

        @pl.loop(0, _NCH)
        def _(j):
            pltpu.async_copy(h_hbm.at[src_v.at[j]], rows_v, sem).wait()
            pltpu.sync_copy(rows_v, agg_s.at[dst_v.at[j]], add=True)

        plsc.subcore_barrier()
        pltpu.sync_copy(agg_s.at[pl.ds(s * _ZR, _ZR)],
                        out_hbm.at[c, pl.ds(s * _ZR, _ZR)])

    return k(h, srcr, dstr, zeros)


def _mlp(h, p0, p1, Wa, ba, Wb, bb, relu_out):
    R = 1000

    def body(h_ref, p0_ref, p1_ref, wa_ref, ba_ref, wb_ref, bb_ref, o_ref):
        m = h_ref[...] + p0_ref[...] + p1_ref[...]
        hid = jnp.dot(m, wa_ref[...], preferred_element_type=jnp.float32) + ba_ref[...]
        hid = jnp.maximum(hid, 0.0)
        o = jnp.dot(hid, wb_ref[...], preferred_element_type=jnp.float32) + bb_ref[...]
        if relu_out:
            o = jnp.maximum(o, 0.0)
        o_ref[...] = o

    bs_rows = pl.BlockSpec((R, _D), lambda i: (i, 0))
    bs_w = pl.BlockSpec((_D, _D), lambda i: (0, 0))
    bs_b = pl.BlockSpec((1, _D), lambda i: (0, 0))
    return pl.pallas_call(
        body,
        grid=(_N // R,),
        in_specs=[bs_rows, bs_rows, bs_rows, bs_w, bs_b, bs_w, bs_b],
        out_specs=bs_rows,
        out_shape=jax.ShapeDtypeStruct((_N, _D), jnp.float32),
    )(h, p0, p1, Wa, ba.reshape(1, _D), Wb, bb.reshape(1, _D))


def kernel(x, edge_index, W0a, b0a, W0b, b0b, W1a, b1a, W1b, b1b, W2a, b2a, W2b, b2b):
    src = edge_index[0]
    dst = edge_index[1]
    pad = _EP - _E
    srcr = jnp.concatenate([src, jnp.zeros((pad,), jnp.int32)]).reshape(_NW, _NCH, _BE)
    dstr = jnp.concatenate([dst, jnp.full((pad,), _N, jnp.int32)]).reshape(_NW, _NCH, _BE)
    zeros = jnp.zeros((_NPAD, _D), jnp.float32)

    h = x
    for Wa, ba, Wb, bb, relu_out in (
        (W0a, b0a, W0b, b0b, True),
        (W1a, b1a, W1b, b1b, True),
        (W2a, b2a, W2b, b2b, False),
    ):
        parts = _segment_sum_partials(h, srcr, dstr, zeros)
        h = _mlp(h, parts[0, :_N], parts[1, :_N], Wa, ba, Wb, bb, relu_out)
    return h


# speedup vs baseline: 3.1066x; 1.1362x over previous
import functools

import jax
import jax.numpy as jnp
from jax import lax
from jax.experimental import pallas as pl
from jax.experimental.pallas import tpu as pltpu
from jax.experimental.pallas import tpu_sc as plsc

_N = 10000
_D = 128
_E = 320000
_NC = 2
_NS = 16
_NW = _NC * _NS
_BE = 64
_NCH = 160
_HCH = 40
_EPW = _NCH * _BE
_EP = _NW * _EPW
_ZR = 640
_NPAD = _NS * _ZR


def _segment_sum_partials(h, srcr, dstr, zeros):
    mesh = plsc.VectorSubcoreMesh(core_axis_name="c", subcore_axis_name="s")

    @functools.partial(
        pl.kernel,
        out_type=jax.ShapeDtypeStruct((_NC, _NPAD, _D), jnp.float32),
        mesh=mesh,
        scratch_types=[
            pltpu.VMEM((_HCH, _BE), jnp.int32),
            pltpu.VMEM((_HCH, _BE), jnp.int32),
            pltpu.VMEM((_BE, _D), jnp.float32),
            pltpu.VMEM((_BE, _D), jnp.float32),
            pltpu.VMEM((_BE, _D), jnp.float32),
            pltpu.VMEM((_BE, _D), jnp.float32),
            pltpu.VMEM_SHARED((_NPAD, _D), jnp.float32),
            pltpu.SemaphoreType.DMA,
            pltpu.SemaphoreType.DMA,
            pltpu.SemaphoreType.DMA,
            pltpu.SemaphoreType.DMA,
            pltpu.SemaphoreType.DMA,
            pltpu.SemaphoreType.DMA,
            pltpu.SemaphoreType.DMA,
            pltpu.SemaphoreType.DMA,
            pltpu.SemaphoreType.DMA,
        ],
    )
    def k(h_hbm, src_hbm, dst_hbm, zero_hbm, out_hbm, src_v, dst_v,
          rows0, rows1, rows2, rows3, agg_s, sem0,
          gs0, gs1, gs2, gs3, as0, as1, as2, as3):
        c = lax.axis_index("c")
        s = lax.axis_index("s")
        wid = s * _NC + c
        bufs = (rows0, rows1, rows2, rows3)
        gsem = (gs0, gs1, gs2, gs3)
        asem = (as0, as1, as2, as3)

        def stage_idx(half):
            ssrc = src_hbm.at[wid, pl.ds(half * _HCH, _HCH)]
            sdst = dst_hbm.at[wid, pl.ds(half * _HCH, _HCH)]
            pltpu.async_copy(ssrc, src_v, gs0)
            pltpu.async_copy(sdst, dst_v, as0)
            pltpu.make_async_copy(ssrc, src_v, gs0).wait()
            pltpu.make_async_copy(sdst, dst_v, as0).wait()

        zsrc = zero_hbm.at[pl.ds(s * _ZR, _ZR)]
        zdst = agg_s.at[pl.ds(s * _ZR, _ZR)]
        pltpu.async_copy(zsrc, zdst, sem0)
        stage_idx(0)
        pltpu.make_async_copy(zsrc, zdst, sem0).wait()
        plsc.subcore_barrier()

        def fire_gather(jj, b):
            pltpu.async_copy(h_hbm.at[src_v.at[jj]], bufs[b], gsem[b])

        def gather_done(jj, b):
            pltpu.make_async_copy(h_hbm.at[src_v.at[jj]], bufs[b], gsem[b]).wait()

        def fire_add(jj, b):
            pltpu.async_copy(bufs[b], agg_s.at[dst_v.at[jj]], asem[b], add=True)

        def add_done(jj, b):
            pltpu.make_async_copy(bufs[b], agg_s.at[dst_v.at[jj]], asem[b]).wait()

        def run_half():
            fire_gather(0, 0)
            fire_gather(1, 1)
            fire_gather(2, 2)
            fire_gather(3, 3)

            @pl.loop(0, _HCH, step=4)
            def _(j):
                for b in range(2):
                    gather_done(j + b, b)
                    fire_add(j + b, b)
                for b in range(2):
                    add_done(j + b, b)

                @pl.when(j + 4 < _HCH)
                def _():
                    fire_gather(j + 4, 0)
                    fire_gather(j + 5, 1)

                for b in range(2):
                    gather_done(j + 2 + b, 2 + b)
                    fire_add(j + 2 + b, 2 + b)
                for b in range(2):
                    add_done(j + 2 + b, 2 + b)

                @pl.when(j + 6 < _HCH)
                def _():
                    fire_gather(j + 6, 2)
                    fire_gather(j + 7, 3)

        run_half()
        for q in range(1, _NCH // _HCH):
            stage_idx(q)
            run_half()
        plsc.subcore_barrier()
        pltpu.sync_copy(agg_s.at[pl.ds(s * _ZR, _ZR)],
                        out_hbm.at[c, pl.ds(s * _ZR, _ZR)])

    return k(h, srcr, dstr, zeros)


def _mlp(h, p0, p1, Wa, ba, Wb, bb, relu_out):
    R = 1000

    def body(h_ref, p0_ref, p1_ref, wa_ref, ba_ref, wb_ref, bb_ref, o_ref):
        m = h_ref[...] + p0_ref[...] + p1_ref[...]
        hid = jnp.dot(m, wa_ref[...], preferred_element_type=jnp.float32) + ba_ref[...]
        hid = jnp.maximum(hid, 0.0)
        o = jnp.dot(hid, wb_ref[...], preferred_element_type=jnp.float32) + bb_ref[...]
        if relu_out:
            o = jnp.maximum(o, 0.0)
        o_ref[...] = o

    bs_rows = pl.BlockSpec((R, _D), lambda i: (i, 0))
    bs_w = pl.BlockSpec((_D, _D), lambda i: (0, 0))
    bs_b = pl.BlockSpec((1, _D), lambda i: (0, 0))
    return pl.pallas_call(
        body,
        grid=(_N // R,),
        in_specs=[bs_rows, bs_rows, bs_rows, bs_w, bs_b, bs_w, bs_b],
        out_specs=bs_rows,
        out_shape=jax.ShapeDtypeStruct((_N, _D), jnp.float32),
    )(h, p0, p1, Wa, ba.reshape(1, _D), Wb, bb.reshape(1, _D))


def kernel(x, edge_index, W0a, b0a, W0b, b0b, W1a, b1a, W1b, b1b, W2a, b2a, W2b, b2b):
    src = edge_index[0]
    dst = edge_index[1]
    pad = _EP - _E
    srcr = jnp.concatenate([src, jnp.zeros((pad,), jnp.int32)]).reshape(_NW, _NCH, _BE)
    dstr = jnp.concatenate([dst, jnp.full((pad,), _N, jnp.int32)]).reshape(_NW, _NCH, _BE)
    zeros = jnp.zeros((_NPAD, _D), jnp.float32)

    h = x
    for Wa, ba, Wb, bb, relu_out in (
        (W0a, b0a, W0b, b0b, True),
        (W1a, b1a, W1b, b1b, True),
        (W2a, b2a, W2b, b2b, False),
    ):
        parts = _segment_sum_partials(h, srcr, dstr, zeros)
        h = _mlp(h, parts[0, :_N], parts[1, :_N], Wa, ba, Wb, bb, relu_out)
    return h


# speedup vs baseline: 10.3283x; 3.3247x over previous
import functools

import jax
import jax.numpy as jnp
from jax import lax
from jax.experimental import pallas as pl
from jax.experimental.pallas import tpu as pltpu
from jax.experimental.pallas import tpu_sc as plsc

_N = 10000
_D = 128
_E = 320000
_NC = 2
_NS = 16
_NW = _NC * _NS
_BE = 64
_NCH = 160
_HCH = 40
_EPW = _NCH * _BE
_EP = _NW * _EPW
_ZR = 640
_NPAD = _NS * _ZR


def _segment_sum_partials(h, srcr, dstr, zeros):
    mesh = plsc.VectorSubcoreMesh(core_axis_name="c", subcore_axis_name="s")

    @functools.partial(
        pl.kernel,
        out_type=jax.ShapeDtypeStruct((_NC, _NPAD, _D), jnp.float32),
        mesh=mesh,
        scratch_types=[
            pltpu.VMEM((_HCH, _BE), jnp.int32),
            pltpu.VMEM((_HCH, _BE), jnp.int32),
            pltpu.VMEM((_BE, _D), jnp.float32),
            pltpu.VMEM((_BE, _D), jnp.float32),
            pltpu.VMEM((_BE, _D), jnp.float32),
            pltpu.VMEM((_BE, _D), jnp.float32),
            pltpu.VMEM_SHARED((_NPAD, _D), jnp.float32),
            pltpu.SemaphoreType.DMA,
            pltpu.SemaphoreType.DMA,
            pltpu.SemaphoreType.DMA,
            pltpu.SemaphoreType.DMA,
            pltpu.SemaphoreType.DMA,
            pltpu.SemaphoreType.DMA,
            pltpu.SemaphoreType.DMA,
            pltpu.SemaphoreType.DMA,
            pltpu.SemaphoreType.DMA,
        ],
    )
    def k(h_hbm, src_hbm, dst_hbm, zero_hbm, out_hbm, src_v, dst_v,
          rows0, rows1, rows2, rows3, agg_s, sem0,
          gs0, gs1, gs2, gs3, as0, as1, as2, as3):
        c = lax.axis_index("c")
        s = lax.axis_index("s")
        wid = s * _NC + c
        bufs = (rows0, rows1, rows2, rows3)
        gsem = (gs0, gs1, gs2, gs3)
        asem = (as0, as1, as2, as3)

        def stage_idx(half):
            ssrc = src_hbm.at[wid, pl.ds(half * _HCH, _HCH)]
            sdst = dst_hbm.at[wid, pl.ds(half * _HCH, _HCH)]
            pltpu.async_copy(ssrc, src_v, gs0)
            pltpu.async_copy(sdst, dst_v, as0)
            pltpu.make_async_copy(ssrc, src_v, gs0).wait()
            pltpu.make_async_copy(sdst, dst_v, as0).wait()

        zsrc = zero_hbm.at[pl.ds(s * _ZR, _ZR)]
        zdst = agg_s.at[pl.ds(s * _ZR, _ZR)]
        pltpu.async_copy(zsrc, zdst, sem0)
        stage_idx(0)
        pltpu.make_async_copy(zsrc, zdst, sem0).wait()
        plsc.subcore_barrier()

        def fire_gather(jj, b):
            pltpu.async_copy(h_hbm.at[src_v.at[jj]], bufs[b], gsem[b])

        def gather_done(jj, b):
            pltpu.make_async_copy(h_hbm.at[src_v.at[jj]], bufs[b], gsem[b]).wait()

        def fire_add(jj, b):
            pltpu.async_copy(bufs[b], agg_s.at[dst_v.at[jj]], asem[b], add=True)

        def add_done(jj, b):
            pltpu.make_async_copy(bufs[b], agg_s.at[dst_v.at[jj]], asem[b]).wait()

        def run_half():
            fire_gather(0, 0)
            fire_gather(1, 1)
            fire_gather(2, 2)
            fire_gather(3, 3)

            @pl.loop(0, _HCH, step=4)
            def _(j):
                for b in range(2):
                    gather_done(j + b, b)
                    fire_add(j + b, b)
                for b in range(2):
                    add_done(j + b, b)

                @pl.when(j + 4 < _HCH)
                def _():
                    fire_gather(j + 4, 0)
                    fire_gather(j + 5, 1)

                for b in range(2):
                    gather_done(j + 2 + b, 2 + b)
                    fire_add(j + 2 + b, 2 + b)
                for b in range(2):
                    add_done(j + 2 + b, 2 + b)

                @pl.when(j + 6 < _HCH)
                def _():
                    fire_gather(j + 6, 2)
                    fire_gather(j + 7, 3)

        run_half()
        for q in range(1, _NCH // _HCH):
            stage_idx(q)
            run_half()
        plsc.subcore_barrier()
        pltpu.sync_copy(agg_s.at[pl.ds(s * _ZR, _ZR)],
                        out_hbm.at[c, pl.ds(s * _ZR, _ZR)])

    return k(h, srcr, dstr, zeros)


def _mlp(h, p0, p1, Wa, ba, Wb, bb, relu_out):
    R = 1000

    def body(h_ref, p0_ref, p1_ref, wa_ref, ba_ref, wb_ref, bb_ref, o_ref):
        m = h_ref[...] + p0_ref[...] + p1_ref[...]
        hid = jnp.dot(m, wa_ref[...], preferred_element_type=jnp.float32) + ba_ref[...]
        hid = jnp.maximum(hid, 0.0)
        o = jnp.dot(hid, wb_ref[...], preferred_element_type=jnp.float32) + bb_ref[...]
        if relu_out:
            o = jnp.maximum(o, 0.0)
        o_ref[...] = o

    bs_rows = pl.BlockSpec((R, _D), lambda i: (i, 0))
    bs_w = pl.BlockSpec((_D, _D), lambda i: (0, 0))
    bs_b = pl.BlockSpec((1, _D), lambda i: (0, 0))
    return pl.pallas_call(
        body,
        grid=(_N // R,),
        in_specs=[bs_rows, bs_rows, bs_rows, bs_w, bs_b, bs_w, bs_b],
        out_specs=bs_rows,
        out_shape=jax.ShapeDtypeStruct((_N, _D), jnp.float32),
    )(h, p0, p1, Wa, ba.reshape(1, _D), Wb, bb.reshape(1, _D))


def kernel(x, edge_index, W0a, b0a, W0b, b0b, W1a, b1a, W1b, b1b, W2a, b2a, W2b, b2b):
    src = edge_index[0]
    dst = edge_index[1]
    pad = _EP - _E
    pad_i = jnp.arange(pad, dtype=jnp.int32)
    srcr = jnp.concatenate([src, pad_i % _N]).reshape(_NW, _NCH, _BE)
    dstr = jnp.concatenate([dst, _N + pad_i % (_NPAD - _N)]).reshape(_NW, _NCH, _BE)
    zeros = jnp.zeros((_NPAD, _D), jnp.float32)

    h = x
    for Wa, ba, Wb, bb, relu_out in (
        (W0a, b0a, W0b, b0b, True),
        (W1a, b1a, W1b, b1b, True),
        (W2a, b2a, W2b, b2b, False),
    ):
        parts = _segment_sum_partials(h, srcr, dstr, zeros)
        h = _mlp(h, parts[0, :_N], parts[1, :_N], Wa, ba, Wb, bb, relu_out)
    return h
